# baseline (speedup 1.0000x reference)
import jax
import jax.numpy as jnp
from jax.experimental import pallas as pl
from jax.experimental.pallas import tpu as pltpu

N = 512
EMB = 64
HD = 64
D_IN = EMB + 2 * HD
MID = 64
BOT = HD

BI = 16

_CONTRACT_MINOR = (((1,), (1,)), ((), ()))


def _pool_body(cxy_ref, nei_ref, lstm_blk_ref, lstm_ref, W1T_ref,
               At_ref, biasT_ref, W2T_ref, b2c_ref, out_ref, BjT_s, H_s):
    k = pl.program_id(0)

    @pl.when(k == 0)
    def _():
        BjT_s[...] = jax.lax.dot_general(
            W1T_ref[:, EMB:EMB + HD], lstm_ref[...], _CONTRACT_MINOR,
            preferred_element_type=jnp.float32) + biasT_ref[...]

    CiT_blk = jax.lax.dot_general(
        W1T_ref[:, EMB + HD:], lstm_blk_ref[...], _CONTRACT_MINOR,
        preferred_element_type=jnp.float32)
    BjT = BjT_s[...]
    A0 = At_ref[:, 0:1]
    A1 = At_ref[:, 1:2]
    for il in range(BI):
        pre = A0 * cxy_ref[0, il:il + 1, :] + A1 * cxy_ref[1, il:il + 1, :]
        H_s[:, il * N:(il + 1) * N] = jnp.maximum(
            pre + BjT + CiT_blk[:, il:il + 1], 0.0).astype(jnp.bfloat16)
    P = jnp.dot(W2T_ref[...], H_s[...],
                preferred_element_type=jnp.float32)
    cols = []
    for il in range(BI):
        masked = jnp.where(nei_ref[il:il + 1, :] > 0,
                           P[:, il * N:(il + 1) * N], -1e30)
        cols.append(jnp.max(masked, axis=1, keepdims=True))
    poolT = jnp.concatenate(cols, axis=1)
    out_ref[...] = jnp.maximum(poolT + b2c_ref[...], 0.0).T


def kernel(corr_index, nei_index, nei_num, lstm_state, curr_pos_abs,
           W_se, b_se, W1, b1, W2, b2):
    cxy = corr_index.transpose(2, 0, 1)
    At = (W_se @ W1[:EMB]).T
    biasT = (b_se @ W1[:EMB] + b1)[:, None]
    W1T = W1.T
    W2Tb = W2.T.astype(jnp.bfloat16)
    b2c = b2[:, None]

    out = pl.pallas_call(
        _pool_body,
        grid=(N // BI,),
        in_specs=[
            pl.BlockSpec((2, BI, N), lambda k: (0, k, 0)),
            pl.BlockSpec((BI, N), lambda k: (k, 0)),
            pl.BlockSpec((BI, HD), lambda k: (k, 0)),
            pl.BlockSpec((N, HD), lambda k: (0, 0)),
            pl.BlockSpec((MID, D_IN), lambda k: (0, 0)),
            pl.BlockSpec((MID, 2), lambda k: (0, 0)),
            pl.BlockSpec((MID, 1), lambda k: (0, 0)),
            pl.BlockSpec((BOT, MID), lambda k: (0, 0)),
            pl.BlockSpec((BOT, 1), lambda k: (0, 0)),
        ],
        out_specs=pl.BlockSpec((BI, BOT), lambda k: (k, 0)),
        out_shape=jax.ShapeDtypeStruct((N, BOT), jnp.float32),
        scratch_shapes=[pltpu.VMEM((MID, N), jnp.float32),
                        pltpu.VMEM((MID, BI * N), jnp.bfloat16)],
    )(cxy, nei_index, lstm_state, lstm_state, W1T, At, biasT, W2Tb, b2c)
    return out


# speedup vs baseline: 1.1460x; 1.1460x over previous
import jax
import jax.numpy as jnp
from jax.experimental import pallas as pl
from jax.experimental.pallas import tpu as pltpu

N = 512
EMB = 64
HD = 64
MID = 64
BOT = HD

BI = 16
KX = 72

_CONTRACT_MINOR = (((1,), (1,)), ((), ()))


def _pool_body(cx_ref, cy_ref, nei_ref, lstm_blk_ref, lstmT_ref, W1iT_ref,
               Vfix_ref, W2T_ref, b2c_ref, out_ref, X0_s, X1_s, H_s):
    k = pl.program_id(0)

    @pl.when(k == 0)
    def _():
        for X in (X0_s, X1_s):
            X[2:2 + HD, :] = lstmT_ref[...].astype(jnp.bfloat16)
            X[2 + HD:3 + HD, :] = jnp.ones((1, N), jnp.bfloat16)
            X[3 + HD:, :] = jnp.zeros((KX - 3 - HD, N), jnp.bfloat16)

    CiT_blk = jax.lax.dot_general(
        W1iT_ref[...], lstm_blk_ref[...], _CONTRACT_MINOR,
        preferred_element_type=jnp.float32)
    Vfix = Vfix_ref[...]
    for il in range(BI):
        X = X0_s if il % 2 == 0 else X1_s
        X[0:1, :] = cx_ref[il:il + 1, :].astype(jnp.bfloat16)
        X[1:2, :] = cy_ref[il:il + 1, :].astype(jnp.bfloat16)
        preF = jnp.dot(Vfix, X[...], preferred_element_type=jnp.float32)
        H_s[:, il * N:(il + 1) * N] = jnp.maximum(
            preF + CiT_blk[:, il:il + 1], 0.0).astype(jnp.bfloat16)
    P = jnp.dot(W2T_ref[...], H_s[...],
                preferred_element_type=jnp.float32)
    cols = []
    for il in range(BI):
        masked = jnp.where(nei_ref[il:il + 1, :] > 0,
                           P[:, il * N:(il + 1) * N], -1e30)
        cols.append(jnp.max(masked, axis=1, keepdims=True))
    poolT = jnp.concatenate(cols, axis=1)
    out_ref[...] = jnp.maximum(poolT + b2c_ref[...], 0.0).T


def kernel(corr_index, nei_index, nei_num, lstm_state, curr_pos_abs,
           W_se, b_se, W1, b1, W2, b2):
    cx = corr_index[:, :, 0]
    cy = corr_index[:, :, 1]
    At = (W_se @ W1[:EMB]).T
    biasT = (b_se @ W1[:EMB] + b1)[:, None]
    W1jT = W1[EMB:EMB + HD].T
    W1iT = W1[EMB + HD:].T
    Vfix = jnp.concatenate(
        [At, W1jT, biasT, jnp.zeros((MID, KX - 3 - HD), jnp.float32)],
        axis=1).astype(jnp.bfloat16)
    W2Tb = W2.T.astype(jnp.bfloat16)
    b2c = b2[:, None]
    lstmT = lstm_state.T

    out = pl.pallas_call(
        _pool_body,
        grid=(N // BI,),
        in_specs=[
            pl.BlockSpec((BI, N), lambda k: (k, 0)),
            pl.BlockSpec((BI, N), lambda k: (k, 0)),
            pl.BlockSpec((BI, N), lambda k: (k, 0)),
            pl.BlockSpec((BI, HD), lambda k: (k, 0)),
            pl.BlockSpec((HD, N), lambda k: (0, 0)),
            pl.BlockSpec((MID, HD), lambda k: (0, 0)),
            pl.BlockSpec((MID, KX), lambda k: (0, 0)),
            pl.BlockSpec((BOT, MID), lambda k: (0, 0)),
            pl.BlockSpec((BOT, 1), lambda k: (0, 0)),
        ],
        out_specs=pl.BlockSpec((BI, BOT), lambda k: (k, 0)),
        out_shape=jax.ShapeDtypeStruct((N, BOT), jnp.float32),
        scratch_shapes=[pltpu.VMEM((KX, N), jnp.bfloat16),
                        pltpu.VMEM((KX, N), jnp.bfloat16),
                        pltpu.VMEM((MID, BI * N), jnp.bfloat16)],
    )(cx, cy, nei_index, lstm_state, lstmT, W1iT, Vfix, W2Tb, b2c)
    return out
